# initial kernel scaffold (unmeasured)
import jax
import jax.numpy as jnp
from jax import lax
from jax.experimental import pallas as pl
from jax.experimental.pallas import tpu as pltpu


def kernel(
    x,
):
    def body(*refs):
        pass

    out_shape = jax.ShapeDtypeStruct(..., jnp.float32)
    return pl.pallas_call(body, out_shape=out_shape)(...)


# baseline (device time: 15581 ns/iter reference)
import jax
import jax.numpy as jnp
from jax import lax
from jax.experimental import pallas as pl
from jax.experimental.pallas import tpu as pltpu

N_DEV = 16


def kernel(x):
    m_per, n = x.shape
    inv = 1.0 / (N_DEV * m_per)

    def body(x_ref, out_ref, acc_ref, send_sems, recv_sems):
        my = lax.axis_index("i")

        barrier_sem = pltpu.get_barrier_semaphore()
        for j in range(1, N_DEV):
            pl.semaphore_signal(
                barrier_sem,
                inc=1,
                device_id=((my + j) % N_DEV,),
                device_id_type=pl.DeviceIdType.MESH,
            )
        pl.semaphore_wait(barrier_sem, N_DEV - 1)

        acc_ref[pl.ds(0, 1), :] = jnp.sum(x_ref[...], axis=0, keepdims=True)

        sends = []
        for j in range(1, N_DEV):
            dst = (my + j) % N_DEV
            slot = N_DEV - j
            rdma = pltpu.make_async_remote_copy(
                src_ref=acc_ref.at[pl.ds(0, 1), :],
                dst_ref=acc_ref.at[pl.ds(slot, 1), :],
                send_sem=send_sems.at[j],
                recv_sem=recv_sems.at[slot],
                device_id=(dst,),
                device_id_type=pl.DeviceIdType.MESH,
            )
            rdma.start()
            sends.append(rdma)

        for k in range(1, N_DEV):
            recv = pltpu.make_async_remote_copy(
                src_ref=acc_ref.at[pl.ds(0, 1), :],
                dst_ref=acc_ref.at[pl.ds(k, 1), :],
                send_sem=send_sems.at[0],
                recv_sem=recv_sems.at[k],
                device_id=(my,),
                device_id_type=pl.DeviceIdType.MESH,
            )
            recv.wait_recv()

        for rdma in sends:
            rdma.wait_send()

        out_ref[...] = jnp.sum(acc_ref[...], axis=0, keepdims=True) * inv

    return pl.pallas_call(
        body,
        out_shape=jax.ShapeDtypeStruct((1, n), jnp.float32),
        in_specs=[pl.BlockSpec(memory_space=pltpu.VMEM)],
        out_specs=pl.BlockSpec(memory_space=pltpu.VMEM),
        scratch_shapes=[
            pltpu.VMEM((N_DEV, n), jnp.float32),
            pltpu.SemaphoreType.DMA((N_DEV,)),
            pltpu.SemaphoreType.DMA((N_DEV,)),
        ],
        compiler_params=pltpu.CompilerParams(collective_id=0),
    )(x)


# device time: 15285 ns/iter; 1.0194x vs baseline; 1.0194x over previous
import jax
import jax.numpy as jnp
from jax import lax
from jax.experimental import pallas as pl
from jax.experimental.pallas import tpu as pltpu

N_DEV = 16
GRID = 16


def kernel(x):
    m_per, n = x.shape
    inv = 1.0 / (N_DEV * m_per)
    bm = m_per // GRID

    def body(x_ref, out_ref, acc_ref, comm_ref, send_sems, recv_sems):
        i = pl.program_id(0)
        my = lax.axis_index("i")
        barrier_sem = pltpu.get_barrier_semaphore()

        @pl.when(i == 0)
        def _():
            for j in range(1, N_DEV):
                pl.semaphore_signal(
                    barrier_sem,
                    inc=1,
                    device_id=((my + j) % N_DEV,),
                    device_id_type=pl.DeviceIdType.MESH,
                )

        blk = jnp.sum(x_ref[...], axis=0, keepdims=True)

        @pl.when(i == 0)
        def _():
            acc_ref[...] = blk

        @pl.when(i > 0)
        def _():
            acc_ref[...] = acc_ref[...] + blk

        @pl.when(i == GRID - 1)
        def _():
            pl.semaphore_wait(barrier_sem, N_DEV - 1)
            comm_ref[pl.ds(0, 1), :] = acc_ref[...]

            sends = []
            for j in range(1, N_DEV):
                dst = (my + j) % N_DEV
                slot = N_DEV - j
                rdma = pltpu.make_async_remote_copy(
                    src_ref=comm_ref.at[pl.ds(0, 1), :],
                    dst_ref=comm_ref.at[pl.ds(slot, 1), :],
                    send_sem=send_sems.at[j],
                    recv_sem=recv_sems.at[slot],
                    device_id=(dst,),
                    device_id_type=pl.DeviceIdType.MESH,
                )
                rdma.start()
                sends.append(rdma)

            for k in range(1, N_DEV):
                recv = pltpu.make_async_remote_copy(
                    src_ref=comm_ref.at[pl.ds(0, 1), :],
                    dst_ref=comm_ref.at[pl.ds(k, 1), :],
                    send_sem=send_sems.at[0],
                    recv_sem=recv_sems.at[k],
                    device_id=(my,),
                    device_id_type=pl.DeviceIdType.MESH,
                )
                recv.wait_recv()

            for rdma in sends:
                rdma.wait_send()

            out_ref[...] = jnp.sum(comm_ref[...], axis=0, keepdims=True) * inv

    return pl.pallas_call(
        body,
        grid=(GRID,),
        out_shape=jax.ShapeDtypeStruct((1, n), jnp.float32),
        in_specs=[
            pl.BlockSpec((bm, n), lambda i: (i, 0), memory_space=pltpu.VMEM)
        ],
        out_specs=pl.BlockSpec((1, n), lambda i: (0, 0), memory_space=pltpu.VMEM),
        scratch_shapes=[
            pltpu.VMEM((1, n), jnp.float32),
            pltpu.VMEM((N_DEV, n), jnp.float32),
            pltpu.SemaphoreType.DMA((N_DEV,)),
            pltpu.SemaphoreType.DMA((N_DEV,)),
        ],
        compiler_params=pltpu.CompilerParams(collective_id=0),
    )(x)


# device time: 14926 ns/iter; 1.0439x vs baseline; 1.0241x over previous
import jax
import jax.numpy as jnp
from jax import lax
from jax.experimental import pallas as pl
from jax.experimental.pallas import tpu as pltpu

N_DEV = 16
GRID = 4


def kernel(x):
    m_per, n = x.shape
    inv = 1.0 / (N_DEV * m_per)
    bm = m_per // GRID

    def body(x_ref, out_ref, acc_ref, comm_ref, send_sems, recv_sems):
        i = pl.program_id(0)
        my = lax.axis_index("i")
        barrier_sem = pltpu.get_barrier_semaphore()

        @pl.when(i == 0)
        def _():
            for j in range(1, N_DEV):
                pl.semaphore_signal(
                    barrier_sem,
                    inc=1,
                    device_id=((my + j) % N_DEV,),
                    device_id_type=pl.DeviceIdType.MESH,
                )

        blk = jnp.sum(x_ref[...], axis=0, keepdims=True)

        @pl.when(i == 0)
        def _():
            acc_ref[...] = blk

        @pl.when(i > 0)
        def _():
            acc_ref[...] = acc_ref[...] + blk

        @pl.when(i == GRID - 1)
        def _():
            pl.semaphore_wait(barrier_sem, N_DEV - 1)
            comm_ref[pl.ds(0, 1), :] = acc_ref[...]

            sends = []
            for j in range(1, N_DEV):
                dst = (my + j) % N_DEV
                slot = N_DEV - j
                rdma = pltpu.make_async_remote_copy(
                    src_ref=comm_ref.at[pl.ds(0, 1), :],
                    dst_ref=comm_ref.at[pl.ds(slot, 1), :],
                    send_sem=send_sems.at[j],
                    recv_sem=recv_sems.at[slot],
                    device_id=(dst,),
                    device_id_type=pl.DeviceIdType.MESH,
                )
                rdma.start()
                sends.append(rdma)

            for k in range(1, N_DEV):
                recv = pltpu.make_async_remote_copy(
                    src_ref=comm_ref.at[pl.ds(0, 1), :],
                    dst_ref=comm_ref.at[pl.ds(k, 1), :],
                    send_sem=send_sems.at[0],
                    recv_sem=recv_sems.at[k],
                    device_id=(my,),
                    device_id_type=pl.DeviceIdType.MESH,
                )
                recv.wait_recv()

            for rdma in sends:
                rdma.wait_send()

            out_ref[...] = jnp.sum(comm_ref[...], axis=0, keepdims=True) * inv

    return pl.pallas_call(
        body,
        grid=(GRID,),
        out_shape=jax.ShapeDtypeStruct((1, n), jnp.float32),
        in_specs=[
            pl.BlockSpec((bm, n), lambda i: (i, 0), memory_space=pltpu.VMEM)
        ],
        out_specs=pl.BlockSpec((1, n), lambda i: (0, 0), memory_space=pltpu.VMEM),
        scratch_shapes=[
            pltpu.VMEM((1, n), jnp.float32),
            pltpu.VMEM((N_DEV, n), jnp.float32),
            pltpu.SemaphoreType.DMA((N_DEV,)),
            pltpu.SemaphoreType.DMA((N_DEV,)),
        ],
        compiler_params=pltpu.CompilerParams(collective_id=0),
    )(x)
